# Initial kernel scaffold; baseline (speedup 1.0000x reference)
import jax
import jax.numpy as jnp
from jax.experimental import pallas as pl


def kernel(ent_embeds, rel_embeds, edge_index, edge_rel, target_idx, global_emb, W_msg, W_self, fc_W):
    raise NotImplementedError("write your pallas kernel here")


# speedup vs baseline: 3.4674x; 3.4674x over previous
import functools

import jax
import jax.numpy as jnp
from jax import lax
from jax.experimental import pallas as pl
from jax.experimental.pallas import tpu as pltpu
from jax.experimental.pallas import tpu_sc as plsc

_N = 10000
_E = 320000
_H = 128
_R = 400
_B = 1024
_TBL = _N + _R
_W = 144

_NC = 2
_NS = 16
_EPW = _E // (_NC * _NS)
_C = 80
_NCHUNK = _EPW // _C
_TPS = _B // _NS


def _mm_body(x_ref, w_ref, o_ref):
    o_ref[...] = jnp.dot(x_ref[...], w_ref[...],
                         preferred_element_type=jnp.float32)


def _make_table(stack, w_msg):
    return pl.pallas_call(
        _mm_body,
        grid=(26,),
        in_specs=[
            pl.BlockSpec((400, _H), lambda i: (i, 0)),
            pl.BlockSpec((_H, _H), lambda i: (jnp.where(i < 25, 0, 1), 0)),
        ],
        out_specs=pl.BlockSpec((400, _H), lambda i: (i, 0)),
        out_shape=jax.ShapeDtypeStruct((_TBL, _H), jnp.float32),
    )(stack, w_msg)


def _sc_body(table, srch, dsth, relh, tgth, enth,
             part, enttgt,
             acc, srcv, dstv, relv, abuf, bbuf, mbuf,
             tgtv, sem1, sem2):
    c = lax.axis_index("c")
    s = lax.axis_index("s")
    wid = c * _NS + s
    zero16 = jnp.zeros((16,), jnp.float32)
    one16 = jnp.ones((16,), jnp.float32)

    def _zrow(e, carry):
        for g in range(_W // 16):
            mbuf[e, pl.ds(g * 16, 16)] = zero16
        return carry
    lax.fori_loop(0, _C, _zrow, 0)

    nzchunks = _N // _C
    nmine = (nzchunks - s + _NS - 1) // _NS

    def _zchunk(k, carry):
        pltpu.sync_copy(mbuf, acc.at[pl.ds((s + k * _NS) * _C, _C)])
        return carry
    lax.fori_loop(0, nmine, _zchunk, 0)

    def _orow(e, carry):
        mbuf[e, pl.ds(_H, 16)] = one16
        return carry
    lax.fori_loop(0, _C, _orow, 0)

    plsc.subcore_barrier()

    base = wid * _EPW

    def _chunk(j, carry):
        off = base + j * _C
        pltpu.sync_copy(srch.at[pl.ds(off, _C)], srcv)
        pltpu.sync_copy(dsth.at[pl.ds(off, _C)], dstv)
        pltpu.sync_copy(relh.at[pl.ds(off, _C)], relv)
        ca = pltpu.async_copy(table.at[srcv], abuf, sem1)
        cb = pltpu.async_copy(table.at[relv], bbuf, sem2)
        ca.wait()
        cb.wait()

        def _erow(e, carry2):
            for g in range(_H // 16):
                a = abuf[e, pl.ds(g * 16, 16)]
                b = bbuf[e, pl.ds(g * 16, 16)]
                mbuf[e, pl.ds(g * 16, 16)] = jnp.maximum(a + b, 0.0)
            return carry2
        lax.fori_loop(0, _C, _erow, 0)

        pltpu.sync_copy(mbuf, acc.at[dstv], add=True)
        return carry
    lax.fori_loop(0, _NCHUNK, _chunk, 0)

    plsc.subcore_barrier()

    tb = s * _TPS
    pltpu.sync_copy(tgth.at[pl.ds(tb, _TPS)], tgtv)
    outbuf = mbuf.at[pl.ds(0, _TPS)]
    pltpu.sync_copy(acc.at[tgtv], outbuf)
    pltpu.sync_copy(outbuf, part.at[c, pl.ds(tb, _TPS)])

    @pl.when(c == 0)
    def _():
        entbuf = abuf.at[pl.ds(0, _TPS)]
        pltpu.async_copy(enth.at[tgtv], entbuf, sem1).wait()
        pltpu.sync_copy(entbuf, enttgt.at[pl.ds(tb, _TPS)])


def _sc_edge(table, src, dst, rels, tgt, ent):
    mesh = plsc.VectorSubcoreMesh(core_axis_name="c", subcore_axis_name="s")
    fn = pl.kernel(
        _sc_body,
        out_type=(
            jax.ShapeDtypeStruct((_NC, _B, _W), jnp.float32),
            jax.ShapeDtypeStruct((_B, _H), jnp.float32),
        ),
        mesh=mesh,
        compiler_params=pltpu.CompilerParams(use_tc_tiling_on_sc=False),
        scratch_types=[
            pltpu.VMEM_SHARED((_N, _W), jnp.float32),
            pltpu.VMEM((_C,), jnp.int32),
            pltpu.VMEM((_C,), jnp.int32),
            pltpu.VMEM((_C,), jnp.int32),
            pltpu.VMEM((_C, _H), jnp.float32),
            pltpu.VMEM((_C, _H), jnp.float32),
            pltpu.VMEM((_C, _W), jnp.float32),
            pltpu.VMEM((_TPS,), jnp.int32),
            pltpu.SemaphoreType.DMA,
            pltpu.SemaphoreType.DMA,
        ],
    )
    return fn(table, src, dst, rels, tgt, ent)


def _fin_body(ap_ref, dp_ref, et_ref, ge_ref, ws_ref, fc_ref, o_ref):
    agg = ap_ref[0] + ap_ref[1]
    deg = jnp.maximum(dp_ref[0] + dp_ref[1], 1.0)
    selfloop = jnp.dot(et_ref[...], ws_ref[...],
                       preferred_element_type=jnp.float32)
    h = jnp.maximum(agg / deg + selfloop, 0.0)
    fc_a = fc_ref[:, 0:_H]
    fc_b = fc_ref[:, _H:2 * _H]
    o_ref[...] = (
        lax.dot_general(h, fc_a, (((1,), (1,)), ((), ())),
                        preferred_element_type=jnp.float32)
        + lax.dot_general(ge_ref[...], fc_b, (((1,), (1,)), ((), ())),
                          preferred_element_type=jnp.float32)
    )


def _finalize(ap, dp, enttgt, global_emb, w_self, fc_w):
    return pl.pallas_call(
        _fin_body,
        out_shape=jax.ShapeDtypeStruct((_B, _H), jnp.float32),
    )(ap, dp, enttgt, global_emb, w_self, fc_w)


def kernel(ent_embeds, rel_embeds, edge_index, edge_rel, target_idx,
           global_emb, W_msg, W_self, fc_W):
    stack = jnp.concatenate([ent_embeds, rel_embeds], axis=0)
    table = _make_table(stack, W_msg)
    src = edge_index[0]
    dst = edge_index[1]
    rels = edge_rel + _N
    part, enttgt = _sc_edge(table, src, dst, rels, target_idx, ent_embeds)
    ap = part[:, :, :_H]
    dp = part[:, :, _H:_H + 1]
    return _finalize(ap, dp, enttgt, global_emb, W_self, fc_W)


# speedup vs baseline: 19.6838x; 5.6768x over previous
import functools

import jax
import jax.numpy as jnp
from jax import lax
from jax.experimental import pallas as pl
from jax.experimental.pallas import tpu as pltpu
from jax.experimental.pallas import tpu_sc as plsc

_N = 10000
_E = 320000
_H = 128
_R = 400
_B = 1024
_TBL = _N + _R

_NC = 2
_NS = 16
_EPW = _E // (_NC * _NS)
_SP = 2000
_NSPAN = _EPW // _SP
_C = 48
_CAP = 2096
_AN = 10080
_TPS = _B // _NS
_DUMMY = _N


def _mm_body(x_ref, w_ref, o_ref):
    o_ref[...] = jnp.dot(x_ref[...], w_ref[...],
                         preferred_element_type=jnp.float32)


def _make_table(stack, w_msg):
    return pl.pallas_call(
        _mm_body,
        grid=(26,),
        in_specs=[
            pl.BlockSpec((400, _H), lambda i: (i, 0)),
            pl.BlockSpec((_H, _H), lambda i: (jnp.where(i < 25, 0, 1), 0)),
        ],
        out_specs=pl.BlockSpec((400, _H), lambda i: (i, 0)),
        out_shape=jax.ShapeDtypeStruct((_TBL, _H), jnp.float32),
    )(stack, w_msg)


def _sc_body(table, srch, dsth, relh, tgth, enth,
             part, degpart, enttgt,
             accm, accd,
             rsrc, rdst, rrel, csrc, cdst, crel,
             flagsv, tgtb, abuf, bbuf, onesb, zdeg, degbuf,
             dstv, tgtv, sem1, sem2):
    c = lax.axis_index("c")
    s = lax.axis_index("s")
    wid = c * _NS + s
    zero16 = jnp.zeros((16,), jnp.float32)
    izero16 = jnp.zeros((16,), jnp.int32)
    lanes = lax.iota(jnp.int32, 16)

    def _init(e, carry):
        for g in range(_H // 16):
            abuf[e, pl.ds(g * 16, 16)] = zero16
        zdeg[e, pl.ds(0, 16)] = zero16
        onesb[e, pl.ds(0, 16)] = jnp.ones((16,), jnp.float32)
        return carry
    lax.fori_loop(0, _C, _init, 0)

    nz = _AN // _C
    nmine = (nz - s + _NS - 1) // _NS

    def _zc(k, carry):
        r0 = (s + k * _NS) * _C
        pltpu.sync_copy(abuf, accm.at[pl.ds(r0, _C)])
        pltpu.sync_copy(zdeg, accd.at[pl.ds(r0, _C)])
        return carry
    lax.fori_loop(0, nmine, _zc, 0)

    ione16 = jnp.ones((16,), jnp.int32)

    def _fz(g, carry):
        flagsv[pl.ds(g * 16, 16)] = izero16
        return carry
    lax.fori_loop(0, (_N + 16) // 16, _fz, 0)
    pltpu.sync_copy(tgth, tgtb)

    def _fb(g, carry):
        t16 = tgtb[pl.ds(g * 16, 16)]
        plsc.store_scatter(flagsv, [t16], ione16)
        return carry
    lax.fori_loop(0, _B // 16, _fb, 0)

    plsc.subcore_barrier()

    def _do_chunk(off):
        ca = pltpu.async_copy(table.at[csrc.at[pl.ds(off, _C)]], abuf, sem1)
        cb = pltpu.async_copy(table.at[crel.at[pl.ds(off, _C)]], bbuf, sem2)
        for g in range(_C // 16):
            dstv[pl.ds(g * 16, 16)] = cdst[pl.ds(off + g * 16, 16)]
        ca.wait()
        cb.wait()

        def _erow(e, carry):
            for g in range(_H // 16):
                a = abuf[e, pl.ds(g * 16, 16)]
                b = bbuf[e, pl.ds(g * 16, 16)]
                abuf[e, pl.ds(g * 16, 16)] = jnp.maximum(a + b, 0.0)
            return carry
        lax.fori_loop(0, _C, _erow, 0)
        pltpu.sync_copy(abuf, accm.at[dstv], add=True)
        pltpu.sync_copy(onesb, accd.at[dstv], add=True)

    base = wid * _EPW

    def _span(sp, cnt):
        off = base + sp * _SP
        pltpu.sync_copy(srch.at[pl.ds(off, _SP)], rsrc)
        pltpu.sync_copy(dsth.at[pl.ds(off, _SP)], rdst)
        pltpu.sync_copy(relh.at[pl.ds(off, _SP)], rrel)

        def _scan(g, cnt2):
            d16 = rdst[pl.ds(g * 16, 16)]
            fw = plsc.load_gather(flagsv, [d16])
            keep = fw > 0
            cum = plsc.cumsum(jnp.where(keep, 1, 0))
            pos = cnt2 + cum - 1
            plsc.store_scatter(csrc, [pos], rsrc[pl.ds(g * 16, 16)],
                               mask=keep)
            plsc.store_scatter(cdst, [pos], d16, mask=keep)
            plsc.store_scatter(crel, [pos], rrel[pl.ds(g * 16, 16)],
                               mask=keep)
            return cnt2 + jnp.max(cum)
        cnt = lax.fori_loop(0, _SP // 16, _scan, cnt)

        nproc = cnt // _C

        def _proc(j, carry):
            _do_chunk(j * _C)
            return carry
        lax.fori_loop(0, nproc, _proc, 0)

        @pl.when(nproc > 0)
        def _():
            for g in range(_C // 16):
                v1 = csrc[pl.ds(nproc * _C + g * 16, 16)]
                v2 = cdst[pl.ds(nproc * _C + g * 16, 16)]
                v3 = crel[pl.ds(nproc * _C + g * 16, 16)]
                csrc[pl.ds(g * 16, 16)] = v1
                cdst[pl.ds(g * 16, 16)] = v2
                crel[pl.ds(g * 16, 16)] = v3
        return cnt - nproc * _C

    left = lax.fori_loop(0, _NSPAN, _span, 0)

    @pl.when(left > 0)
    def _():
        dumd = jnp.full((16,), _DUMMY, jnp.int32)
        for g in range(_C // 16):
            p16 = lanes + g * 16
            padm = p16 >= left
            plsc.store_scatter(csrc, [p16], izero16, mask=padm)
            plsc.store_scatter(cdst, [p16], dumd, mask=padm)
            plsc.store_scatter(crel, [p16], dumd, mask=padm)
        _do_chunk(0)

    plsc.subcore_barrier()

    tb = s * _TPS
    pltpu.sync_copy(tgth.at[pl.ds(tb, _TPS)], tgtv)
    pltpu.sync_copy(accd.at[tgtv], degbuf)
    pltpu.sync_copy(degbuf, degpart.at[c, pl.ds(tb, _TPS)])
    for h in range(2):
        tsl = tgtv.at[pl.ds(h * 32, 32)]
        ob = abuf.at[pl.ds(0, 32)]
        pltpu.sync_copy(accm.at[tsl], ob)
        pltpu.sync_copy(ob, part.at[c, pl.ds(tb + h * 32, 32)])

    @pl.when(c == 0)
    def _():
        for h in range(2):
            tsl = tgtv.at[pl.ds(h * 32, 32)]
            eb = bbuf.at[pl.ds(0, 32)]
            pltpu.async_copy(enth.at[tsl], eb, sem1).wait()
            pltpu.sync_copy(eb, enttgt.at[pl.ds(tb + h * 32, 32)])


def _sc_edge(table, src, dst, rels, tgt, ent):
    mesh = plsc.VectorSubcoreMesh(core_axis_name="c", subcore_axis_name="s")
    fn = pl.kernel(
        _sc_body,
        out_type=(
            jax.ShapeDtypeStruct((_NC, _B, _H), jnp.float32),
            jax.ShapeDtypeStruct((_NC, _B, 16), jnp.float32),
            jax.ShapeDtypeStruct((_B, _H), jnp.float32),
        ),
        mesh=mesh,
        compiler_params=pltpu.CompilerParams(use_tc_tiling_on_sc=False,
                                             needs_layout_passes=False),
        scratch_types=[
            pltpu.VMEM_SHARED((_AN, _H), jnp.float32),
            pltpu.VMEM_SHARED((_AN, 16), jnp.float32),
            pltpu.VMEM((_SP,), jnp.int32),
            pltpu.VMEM((_SP,), jnp.int32),
            pltpu.VMEM((_SP,), jnp.int32),
            pltpu.VMEM((_CAP,), jnp.int32),
            pltpu.VMEM((_CAP,), jnp.int32),
            pltpu.VMEM((_CAP,), jnp.int32),
            pltpu.VMEM((_N + 16, ), jnp.int32),
            pltpu.VMEM((_B,), jnp.int32),
            pltpu.VMEM((_C, _H), jnp.float32),
            pltpu.VMEM((_C, _H), jnp.float32),
            pltpu.VMEM((_C, 16), jnp.float32),
            pltpu.VMEM((_C, 16), jnp.float32),
            pltpu.VMEM((_TPS, 16), jnp.float32),
            pltpu.VMEM((_C,), jnp.int32),
            pltpu.VMEM((_TPS,), jnp.int32),
            pltpu.SemaphoreType.DMA,
            pltpu.SemaphoreType.DMA,
        ],
    )
    return fn(table, src, dst, rels, tgt, ent)


def _fin_body(ap_ref, dp_ref, et_ref, ge_ref, ws_ref, fc_ref, o_ref):
    agg = ap_ref[0] + ap_ref[1]
    degs = dp_ref[0] + dp_ref[1]
    deg = jnp.maximum(degs[:, 0:1], 1.0)
    selfloop = jnp.dot(et_ref[...], ws_ref[...],
                       preferred_element_type=jnp.float32)
    h = jnp.maximum(agg / deg + selfloop, 0.0)
    fc_a = fc_ref[:, 0:_H]
    fc_b = fc_ref[:, _H:2 * _H]
    o_ref[...] = (
        lax.dot_general(h, fc_a, (((1,), (1,)), ((), ())),
                        preferred_element_type=jnp.float32)
        + lax.dot_general(ge_ref[...], fc_b, (((1,), (1,)), ((), ())),
                          preferred_element_type=jnp.float32)
    )


def _finalize(ap, dp, enttgt, global_emb, w_self, fc_w):
    return pl.pallas_call(
        _fin_body,
        out_shape=jax.ShapeDtypeStruct((_B, _H), jnp.float32),
    )(ap, dp, enttgt, global_emb, w_self, fc_w)


def kernel(ent_embeds, rel_embeds, edge_index, edge_rel, target_idx,
           global_emb, W_msg, W_self, fc_W):
    stack = jnp.concatenate([ent_embeds, rel_embeds], axis=0)
    table = _make_table(stack, W_msg)
    src = edge_index[0]
    dst = edge_index[1]
    rels = edge_rel + _N
    part, degpart, enttgt = _sc_edge(table, src, dst, rels,
                                     target_idx, ent_embeds)
    return _finalize(part, degpart, enttgt, global_emb, W_self, fc_W)


# speedup vs baseline: 19.6852x; 1.0001x over previous
import functools

import jax
import jax.numpy as jnp
from jax import lax
from jax.experimental import pallas as pl
from jax.experimental.pallas import tpu as pltpu
from jax.experimental.pallas import tpu_sc as plsc

_N = 10000
_E = 320000
_H = 128
_R = 400
_B = 1024
_TBL = _N + _R

_NC = 2
_NS = 16
_EPW = _E // (_NC * _NS)
_SP = 2000
_NSPAN = _EPW // _SP
_C = 48
_CAP = 2096
_AN = 10080
_TPS = _B // _NS
_DUMMY = _N


def _mm_body(x_ref, w_ref, o_ref):
    o_ref[...] = jnp.dot(x_ref[...], w_ref[...],
                         preferred_element_type=jnp.float32)


def _make_table(stack, w_msg):
    return pl.pallas_call(
        _mm_body,
        grid=(26,),
        in_specs=[
            pl.BlockSpec((400, _H), lambda i: (i, 0)),
            pl.BlockSpec((_H, _H), lambda i: (jnp.where(i < 25, 0, 1), 0)),
        ],
        out_specs=pl.BlockSpec((400, _H), lambda i: (i, 0)),
        out_shape=jax.ShapeDtypeStruct((_TBL, _H), jnp.float32),
    )(stack, w_msg)


def _sc_body(table, srch, dsth, relh, tgth, enth,
             part, degpart, enttgt,
             accm, accd,
             rsrc, rdst, rrel, csrc, cdst, crel,
             flagsv, tgtb, abuf, bbuf, onesb, zdeg, degbuf,
             dstv, tgtv, sem1, sem2):
    c = lax.axis_index("c")
    s = lax.axis_index("s")
    wid = c * _NS + s
    zero16 = jnp.zeros((16,), jnp.float32)
    izero16 = jnp.zeros((16,), jnp.int32)
    lanes = lax.iota(jnp.int32, 16)

    with jax.named_scope("p_zero"):
        def _init(e, carry):
            for g in range(_H // 16):
                abuf[e, pl.ds(g * 16, 16)] = zero16
            zdeg[e, pl.ds(0, 16)] = zero16
            onesb[e, pl.ds(0, 16)] = jnp.ones((16,), jnp.float32)
            return carry
        lax.fori_loop(0, _C, _init, 0)

        nz = _AN // _C
        nmine = (nz - s + _NS - 1) // _NS

        def _zc(k, carry):
            r0 = (s + k * _NS) * _C
            pltpu.sync_copy(abuf, accm.at[pl.ds(r0, _C)])
            pltpu.sync_copy(zdeg, accd.at[pl.ds(r0, _C)])
            return carry
        lax.fori_loop(0, nmine, _zc, 0)

    with jax.named_scope("p_flags"):
        ione16 = jnp.ones((16,), jnp.int32)

        def _fz(g, carry):
            flagsv[pl.ds(g * 16, 16)] = izero16
            return carry
        lax.fori_loop(0, (_N + 16) // 16, _fz, 0)
        pltpu.sync_copy(tgth, tgtb)

        def _fb(g, carry):
            t16 = tgtb[pl.ds(g * 16, 16)]
            plsc.store_scatter(flagsv, [t16], ione16)
            return carry
        lax.fori_loop(0, _B // 16, _fb, 0)

    with jax.named_scope("p_barrier1"):
        plsc.subcore_barrier()

    def _do_chunk(off):
        ca = pltpu.async_copy(table.at[csrc.at[pl.ds(off, _C)]], abuf, sem1)
        cb = pltpu.async_copy(table.at[crel.at[pl.ds(off, _C)]], bbuf, sem2)
        for g in range(_C // 16):
            dstv[pl.ds(g * 16, 16)] = cdst[pl.ds(off + g * 16, 16)]
        ca.wait()
        cb.wait()

        def _erow(e, carry):
            for g in range(_H // 16):
                a = abuf[e, pl.ds(g * 16, 16)]
                b = bbuf[e, pl.ds(g * 16, 16)]
                abuf[e, pl.ds(g * 16, 16)] = jnp.maximum(a + b, 0.0)
            return carry
        lax.fori_loop(0, _C, _erow, 0)
        pltpu.sync_copy(abuf, accm.at[dstv], add=True)
        pltpu.sync_copy(onesb, accd.at[dstv], add=True)

    base = wid * _EPW

    def _span(sp, cnt):
        off = base + sp * _SP
        pltpu.sync_copy(srch.at[pl.ds(off, _SP)], rsrc)
        pltpu.sync_copy(dsth.at[pl.ds(off, _SP)], rdst)
        pltpu.sync_copy(relh.at[pl.ds(off, _SP)], rrel)

        def _scan(g, cnt2):
            d16 = rdst[pl.ds(g * 16, 16)]
            fw = plsc.load_gather(flagsv, [d16])
            keep = fw > 0
            cum = plsc.cumsum(jnp.where(keep, 1, 0))
            pos = cnt2 + cum - 1
            plsc.store_scatter(csrc, [pos], rsrc[pl.ds(g * 16, 16)],
                               mask=keep)
            plsc.store_scatter(cdst, [pos], d16, mask=keep)
            plsc.store_scatter(crel, [pos], rrel[pl.ds(g * 16, 16)],
                               mask=keep)
            return cnt2 + jnp.max(cum)
        cnt = lax.fori_loop(0, _SP // 16, _scan, cnt)

        nproc = cnt // _C

        def _proc(j, carry):
            _do_chunk(j * _C)
            return carry
        lax.fori_loop(0, nproc, _proc, 0)

        @pl.when(nproc > 0)
        def _():
            for g in range(_C // 16):
                v1 = csrc[pl.ds(nproc * _C + g * 16, 16)]
                v2 = cdst[pl.ds(nproc * _C + g * 16, 16)]
                v3 = crel[pl.ds(nproc * _C + g * 16, 16)]
                csrc[pl.ds(g * 16, 16)] = v1
                cdst[pl.ds(g * 16, 16)] = v2
                crel[pl.ds(g * 16, 16)] = v3
        return cnt - nproc * _C

    with jax.named_scope("p_edges"):
        left = lax.fori_loop(0, _NSPAN, _span, 0)

    @pl.when(left > 0)
    def _():
        dumd = jnp.full((16,), _DUMMY, jnp.int32)
        for g in range(_C // 16):
            p16 = lanes + g * 16
            padm = p16 >= left
            plsc.store_scatter(csrc, [p16], izero16, mask=padm)
            plsc.store_scatter(cdst, [p16], dumd, mask=padm)
            plsc.store_scatter(crel, [p16], dumd, mask=padm)
        _do_chunk(0)

    with jax.named_scope("p_barrier2"):
        plsc.subcore_barrier()

    tb = s * _TPS
    pltpu.sync_copy(tgth.at[pl.ds(tb, _TPS)], tgtv)
    pltpu.sync_copy(accd.at[tgtv], degbuf)
    pltpu.sync_copy(degbuf, degpart.at[c, pl.ds(tb, _TPS)])
    for h in range(2):
        tsl = tgtv.at[pl.ds(h * 32, 32)]
        ob = abuf.at[pl.ds(0, 32)]
        pltpu.sync_copy(accm.at[tsl], ob)
        pltpu.sync_copy(ob, part.at[c, pl.ds(tb + h * 32, 32)])

    @pl.when(c == 0)
    def _():
        for h in range(2):
            tsl = tgtv.at[pl.ds(h * 32, 32)]
            eb = bbuf.at[pl.ds(0, 32)]
            pltpu.async_copy(enth.at[tsl], eb, sem1).wait()
            pltpu.sync_copy(eb, enttgt.at[pl.ds(tb + h * 32, 32)])


def _sc_edge(table, src, dst, rels, tgt, ent):
    mesh = plsc.VectorSubcoreMesh(core_axis_name="c", subcore_axis_name="s")
    fn = pl.kernel(
        _sc_body,
        out_type=(
            jax.ShapeDtypeStruct((_NC, _B, _H), jnp.float32),
            jax.ShapeDtypeStruct((_NC, _B, 16), jnp.float32),
            jax.ShapeDtypeStruct((_B, _H), jnp.float32),
        ),
        mesh=mesh,
        compiler_params=pltpu.CompilerParams(use_tc_tiling_on_sc=False,
                                             needs_layout_passes=False),
        scratch_types=[
            pltpu.VMEM_SHARED((_AN, _H), jnp.float32),
            pltpu.VMEM_SHARED((_AN, 16), jnp.float32),
            pltpu.VMEM((_SP,), jnp.int32),
            pltpu.VMEM((_SP,), jnp.int32),
            pltpu.VMEM((_SP,), jnp.int32),
            pltpu.VMEM((_CAP,), jnp.int32),
            pltpu.VMEM((_CAP,), jnp.int32),
            pltpu.VMEM((_CAP,), jnp.int32),
            pltpu.VMEM((_N + 16, ), jnp.int32),
            pltpu.VMEM((_B,), jnp.int32),
            pltpu.VMEM((_C, _H), jnp.float32),
            pltpu.VMEM((_C, _H), jnp.float32),
            pltpu.VMEM((_C, 16), jnp.float32),
            pltpu.VMEM((_C, 16), jnp.float32),
            pltpu.VMEM((_TPS, 16), jnp.float32),
            pltpu.VMEM((_C,), jnp.int32),
            pltpu.VMEM((_TPS,), jnp.int32),
            pltpu.SemaphoreType.DMA,
            pltpu.SemaphoreType.DMA,
        ],
    )
    return fn(table, src, dst, rels, tgt, ent)


def _fin_body(ap_ref, dp_ref, et_ref, ge_ref, ws_ref, fc_ref, o_ref):
    agg = ap_ref[0] + ap_ref[1]
    degs = dp_ref[0] + dp_ref[1]
    deg = jnp.maximum(degs[:, 0:1], 1.0)
    selfloop = jnp.dot(et_ref[...], ws_ref[...],
                       preferred_element_type=jnp.float32)
    h = jnp.maximum(agg / deg + selfloop, 0.0)
    fc_a = fc_ref[:, 0:_H]
    fc_b = fc_ref[:, _H:2 * _H]
    o_ref[...] = (
        lax.dot_general(h, fc_a, (((1,), (1,)), ((), ())),
                        preferred_element_type=jnp.float32)
        + lax.dot_general(ge_ref[...], fc_b, (((1,), (1,)), ((), ())),
                          preferred_element_type=jnp.float32)
    )


def _finalize(ap, dp, enttgt, global_emb, w_self, fc_w):
    return pl.pallas_call(
        _fin_body,
        out_shape=jax.ShapeDtypeStruct((_B, _H), jnp.float32),
    )(ap, dp, enttgt, global_emb, w_self, fc_w)


def kernel(ent_embeds, rel_embeds, edge_index, edge_rel, target_idx,
           global_emb, W_msg, W_self, fc_W):
    stack = jnp.concatenate([ent_embeds, rel_embeds], axis=0)
    table = _make_table(stack, W_msg)
    src = edge_index[0]
    dst = edge_index[1]
    rels = edge_rel + _N
    part, degpart, enttgt = _sc_edge(table, src, dst, rels,
                                     target_idx, ent_embeds)
    return _finalize(part, degpart, enttgt, global_emb, W_self, fc_W)


# speedup vs baseline: 24.2946x; 1.2342x over previous
import functools

import jax
import jax.numpy as jnp
from jax import lax
from jax.experimental import pallas as pl
from jax.experimental.pallas import tpu as pltpu
from jax.experimental.pallas import tpu_sc as plsc

_N = 10000
_E = 320000
_H = 128
_R = 400
_B = 1024

_NC = 2
_NS = 16
_EPW = _E // (_NC * _NS)
_SP = 2000
_NSPAN = _EPW // _SP
_C = 48
_CAP = 2096
_AN = 10080
_TPS = _B // _NS
_DUMMY = _N


def _mm_body(x_ref, r_ref, w_ref, o1_ref, o2_ref):
    i = pl.program_id(0)
    o1_ref[...] = jnp.dot(x_ref[...], w_ref[0:_H, :],
                          preferred_element_type=jnp.float32)

    @pl.when(i == 0)
    def _():
        o2_ref[...] = jnp.dot(r_ref[...], w_ref[_H:2 * _H, :],
                              preferred_element_type=jnp.float32)


def _make_tables(ent, rel, w_msg):
    return pl.pallas_call(
        _mm_body,
        grid=(5,),
        in_specs=[
            pl.BlockSpec((2000, _H), lambda i: (i, 0)),
            pl.BlockSpec((_R, _H), lambda i: (0, 0)),
            pl.BlockSpec((2 * _H, _H), lambda i: (0, 0)),
        ],
        out_specs=[
            pl.BlockSpec((2000, _H), lambda i: (i, 0)),
            pl.BlockSpec((_R, _H), lambda i: (0, 0)),
        ],
        out_shape=[jax.ShapeDtypeStruct((_N, _H), jnp.float32),
                   jax.ShapeDtypeStruct((_R, _H), jnp.float32)],
    )(ent, rel, w_msg)


def _sc_body(table1, table2, eidx, relh, tgth, enth,
             part, degpart, enttgt,
             accm, accd,
             rsrc, rdst, rrel, csrc, cdst, crel,
             flagsv, tgtb, abuf, bbuf, onesb, zdeg, degbuf,
             dstv, tgtv, sem1, sem2, sem3):
    c = lax.axis_index("c")
    s = lax.axis_index("s")
    wid = c * _NS + s
    zero16 = jnp.zeros((16,), jnp.float32)
    izero16 = jnp.zeros((16,), jnp.int32)
    lanes = lax.iota(jnp.int32, 16)

    with jax.named_scope("p_zero"):
        def _init(e, carry):
            for g in range(_H // 16):
                abuf[e, pl.ds(g * 16, 16)] = zero16
            zdeg[e, pl.ds(0, 16)] = zero16
            onesb[e, pl.ds(0, 16)] = jnp.ones((16,), jnp.float32)
            return carry
        lax.fori_loop(0, _C, _init, 0)

        nz = _AN // _C
        nmine = (nz - s + _NS - 1) // _NS

        def _zc(k, carry):
            r0 = (s + k * _NS) * _C
            pltpu.async_copy(abuf, accm.at[pl.ds(r0, _C)], sem1)
            pltpu.async_copy(zdeg, accd.at[pl.ds(r0, _C)], sem2)
            return carry
        lax.fori_loop(0, nmine, _zc, 0)

    with jax.named_scope("p_flags"):
        ione16 = jnp.ones((16,), jnp.int32)

        def _fz(g, carry):
            flagsv[pl.ds(g * 16, 16)] = izero16
            return carry
        lax.fori_loop(0, (_N + 16) // 16, _fz, 0)
        pltpu.sync_copy(tgth, tgtb)

        def _fb(g, carry):
            t16 = tgtb[pl.ds(g * 16, 16)]
            plsc.store_scatter(flagsv, [t16], ione16)
            return carry
        lax.fori_loop(0, _B // 16, _fb, 0)

        def _zd(k, carry):
            r0 = (s + k * _NS) * _C
            pltpu.make_async_copy(abuf, accm.at[pl.ds(r0, _C)], sem1).wait()
            pltpu.make_async_copy(zdeg, accd.at[pl.ds(r0, _C)], sem2).wait()
            return carry
        lax.fori_loop(0, nmine, _zd, 0)

    with jax.named_scope("p_barrier1"):
        plsc.subcore_barrier()

    def _do_chunk(off):
        ca = pltpu.async_copy(table1.at[csrc.at[pl.ds(off, _C)]], abuf, sem1)
        cb = pltpu.async_copy(table2.at[crel.at[pl.ds(off, _C)]], bbuf, sem2)
        for g in range(_C // 16):
            dstv[pl.ds(g * 16, 16)] = cdst[pl.ds(off + g * 16, 16)]
        ca.wait()
        cb.wait()

        def _erow(e, carry):
            for g in range(_H // 16):
                a = abuf[e, pl.ds(g * 16, 16)]
                b = bbuf[e, pl.ds(g * 16, 16)]
                abuf[e, pl.ds(g * 16, 16)] = jnp.maximum(a + b, 0.0)
            return carry
        lax.fori_loop(0, _C, _erow, 0)
        pltpu.sync_copy(abuf, accm.at[dstv], add=True)
        pltpu.sync_copy(onesb, accd.at[dstv], add=True)

    base = wid * _EPW

    def _span(sp, cntv):
        off = base + sp * _SP
        pltpu.sync_copy(eidx.at[0, pl.ds(off, _SP)], rsrc)
        pltpu.sync_copy(eidx.at[1, pl.ds(off, _SP)], rdst)
        pltpu.sync_copy(relh.at[pl.ds(off, _SP)], rrel)

        def _scan(g, cv):
            d16 = rdst[pl.ds(g * 16, 16)]
            fw = plsc.load_gather(flagsv, [d16])
            keep = fw > 0
            cum = plsc.cumsum(jnp.where(keep, 1, 0))
            pos = cv + cum - 1
            plsc.store_scatter(csrc, [pos], rsrc[pl.ds(g * 16, 16)],
                               mask=keep)
            plsc.store_scatter(cdst, [pos], d16, mask=keep)
            plsc.store_scatter(crel, [pos], rrel[pl.ds(g * 16, 16)],
                               mask=keep)
            return cv + plsc.all_reduce_population_count(keep)
        cntv = lax.fori_loop(0, _SP // 16, _scan, cntv)

        cnt = jnp.max(cntv)
        nproc = cnt // _C

        def _proc(j, carry):
            _do_chunk(j * _C)
            return carry
        lax.fori_loop(0, nproc, _proc, 0)

        @pl.when(nproc > 0)
        def _():
            for g in range(_C // 16):
                v1 = csrc[pl.ds(nproc * _C + g * 16, 16)]
                v2 = cdst[pl.ds(nproc * _C + g * 16, 16)]
                v3 = crel[pl.ds(nproc * _C + g * 16, 16)]
                csrc[pl.ds(g * 16, 16)] = v1
                cdst[pl.ds(g * 16, 16)] = v2
                crel[pl.ds(g * 16, 16)] = v3
        return cntv - nproc * _C

    with jax.named_scope("p_edges"):
        leftv = lax.fori_loop(0, _NSPAN, _span,
                              jnp.zeros((16,), jnp.int32))
        left = jnp.max(leftv)

        @pl.when(left > 0)
        def _():
            dumd = jnp.full((16,), _DUMMY, jnp.int32)
            for g in range(_C // 16):
                p16 = lanes + g * 16
                padm = p16 >= left
                plsc.store_scatter(csrc, [p16], izero16, mask=padm)
                plsc.store_scatter(cdst, [p16], dumd, mask=padm)
                plsc.store_scatter(crel, [p16], izero16, mask=padm)
            _do_chunk(0)

    with jax.named_scope("p_barrier2"):
        plsc.subcore_barrier()

    with jax.named_scope("p_out"):
        tb = s * _TPS
        pltpu.sync_copy(tgth.at[pl.ds(tb, _TPS)], tgtv)
        t0 = tgtv.at[pl.ds(0, 32)]
        t1 = tgtv.at[pl.ds(32, 32)]
        ob0 = abuf.at[pl.ds(0, 32)]
        ob1 = bbuf.at[pl.ds(0, 32)]
        gd = pltpu.async_copy(accd.at[tgtv], degbuf, sem1)
        g0 = pltpu.async_copy(accm.at[t0], ob0, sem2)
        g1 = pltpu.async_copy(accm.at[t1], ob1, sem3)
        gd.wait()
        wd = pltpu.async_copy(degbuf, degpart.at[c, pl.ds(tb, _TPS)], sem1)
        g0.wait()
        w0 = pltpu.async_copy(ob0, part.at[c, pl.ds(tb, 32)], sem2)
        g1.wait()
        w1 = pltpu.async_copy(ob1, part.at[c, pl.ds(tb + 32, 32)], sem3)
        wd.wait()
        w0.wait()
        w1.wait()

        @pl.when(c == 0)
        def _():
            for h in range(2):
                tsl = tgtv.at[pl.ds(h * 32, 32)]
                eb = abuf.at[pl.ds(0, 32)]
                pltpu.async_copy(enth.at[tsl], eb, sem1).wait()
                pltpu.sync_copy(eb, enttgt.at[pl.ds(tb + h * 32, 32)])


def _sc_edge(table1, table2, eidx, rels, tgt, ent):
    mesh = plsc.VectorSubcoreMesh(core_axis_name="c", subcore_axis_name="s")
    fn = pl.kernel(
        _sc_body,
        out_type=(
            jax.ShapeDtypeStruct((_NC, _B, _H), jnp.float32),
            jax.ShapeDtypeStruct((_NC, _B, 16), jnp.float32),
            jax.ShapeDtypeStruct((_B, _H), jnp.float32),
        ),
        mesh=mesh,
        compiler_params=pltpu.CompilerParams(use_tc_tiling_on_sc=False,
                                             needs_layout_passes=False),
        scratch_types=[
            pltpu.VMEM_SHARED((_AN, _H), jnp.float32),
            pltpu.VMEM_SHARED((_AN, 16), jnp.float32),
            pltpu.VMEM((_SP,), jnp.int32),
            pltpu.VMEM((_SP,), jnp.int32),
            pltpu.VMEM((_SP,), jnp.int32),
            pltpu.VMEM((_CAP,), jnp.int32),
            pltpu.VMEM((_CAP,), jnp.int32),
            pltpu.VMEM((_CAP,), jnp.int32),
            pltpu.VMEM((_N + 16, ), jnp.int32),
            pltpu.VMEM((_B,), jnp.int32),
            pltpu.VMEM((_C, _H), jnp.float32),
            pltpu.VMEM((_C, _H), jnp.float32),
            pltpu.VMEM((_C, 16), jnp.float32),
            pltpu.VMEM((_C, 16), jnp.float32),
            pltpu.VMEM((_TPS, 16), jnp.float32),
            pltpu.VMEM((_C,), jnp.int32),
            pltpu.VMEM((_TPS,), jnp.int32),
            pltpu.SemaphoreType.DMA,
            pltpu.SemaphoreType.DMA,
            pltpu.SemaphoreType.DMA,
        ],
    )
    return fn(table1, table2, eidx, rels, tgt, ent)


def _fin_body(ap_ref, dp_ref, et_ref, ge_ref, ws_ref, fc_ref, o_ref):
    agg = ap_ref[0] + ap_ref[1]
    degs = dp_ref[0] + dp_ref[1]
    deg = jnp.maximum(degs[:, 0:1], 1.0)
    selfloop = jnp.dot(et_ref[...], ws_ref[...],
                       preferred_element_type=jnp.float32)
    h = jnp.maximum(agg / deg + selfloop, 0.0)
    fc_a = fc_ref[:, 0:_H]
    fc_b = fc_ref[:, _H:2 * _H]
    o_ref[...] = (
        lax.dot_general(h, fc_a, (((1,), (1,)), ((), ())),
                        preferred_element_type=jnp.float32)
        + lax.dot_general(ge_ref[...], fc_b, (((1,), (1,)), ((), ())),
                          preferred_element_type=jnp.float32)
    )


def _finalize(ap, dp, enttgt, global_emb, w_self, fc_w):
    return pl.pallas_call(
        _fin_body,
        out_shape=jax.ShapeDtypeStruct((_B, _H), jnp.float32),
    )(ap, dp, enttgt, global_emb, w_self, fc_w)


def kernel(ent_embeds, rel_embeds, edge_index, edge_rel, target_idx,
           global_emb, W_msg, W_self, fc_W):
    table1, table2 = _make_tables(ent_embeds, rel_embeds, W_msg)
    part, degpart, enttgt = _sc_edge(table1, table2, edge_index, edge_rel,
                                     target_idx, ent_embeds)
    return _finalize(part, degpart, enttgt, global_emb, W_self, fc_W)


# speedup vs baseline: 26.5658x; 1.0935x over previous
import functools

import jax
import jax.numpy as jnp
from jax import lax
from jax.experimental import pallas as pl
from jax.experimental.pallas import tpu as pltpu
from jax.experimental.pallas import tpu_sc as plsc

_N = 10000
_E = 320000
_H = 128
_R = 400
_B = 1024

_NC = 2
_NS = 16
_EPW = _E // (_NC * _NS)
_SP = 2000
_NSPAN = _EPW // _SP
_NSUP = 5
_C = 32
_CAP = 2048
_MASK = _CAP - 1
_THR = _CAP - 400 - 2 * _C
_AN = 10016
_TPS = _B // _NS
_DUMMY = _N


def _mm_body(x_ref, r_ref, w_ref, o1_ref, o2_ref):
    i = pl.program_id(0)
    o1_ref[...] = jnp.dot(x_ref[...], w_ref[0:_H, :],
                          preferred_element_type=jnp.float32)

    @pl.when(i == 0)
    def _():
        o2_ref[...] = jnp.dot(r_ref[...], w_ref[_H:2 * _H, :],
                              preferred_element_type=jnp.float32)


def _make_tables(ent, rel, w_msg):
    return pl.pallas_call(
        _mm_body,
        grid=(5,),
        in_specs=[
            pl.BlockSpec((2000, _H), lambda i: (i, 0)),
            pl.BlockSpec((_R, _H), lambda i: (0, 0)),
            pl.BlockSpec((2 * _H, _H), lambda i: (0, 0)),
        ],
        out_specs=[
            pl.BlockSpec((2000, _H), lambda i: (i, 0)),
            pl.BlockSpec((_R, _H), lambda i: (0, 0)),
        ],
        out_shape=[jax.ShapeDtypeStruct((_N, _H), jnp.float32),
                   jax.ShapeDtypeStruct((_R, _H), jnp.float32)],
    )(ent, rel, w_msg)


def _sc_body(table1, table2, eidx, relh, tgth, enth,
             part, degpart, enttgt,
             accm, accd,
             rsrc, rdst, rrel, csrc, cdst, crel,
             flagsv, tgtb, abuf0, abuf1, bbuf0, bbuf1,
             onesb, zdeg, dstv0, dstv1, tgtv,
             sem1, sem2, sem3, sem4):
    c = lax.axis_index("c")
    s = lax.axis_index("s")
    wid = c * _NS + s
    zero16 = jnp.zeros((16,), jnp.float32)
    izero16 = jnp.zeros((16,), jnp.int32)
    lanes = lax.iota(jnp.int32, 16)
    abufs = (abuf0, abuf1)
    bbufs = (bbuf0, bbuf1)
    dstvs = (dstv0, dstv1)
    gsems = (sem1, sem2)

    with jax.named_scope("p_zero"):
        def _init(e, carry):
            for g in range(_H // 16):
                abuf0[e, pl.ds(g * 16, 16)] = zero16
            zdeg[e, pl.ds(0, 16)] = zero16
            onesb[e, pl.ds(0, 16)] = jnp.ones((16,), jnp.float32)
            return carry
        lax.fori_loop(0, _C, _init, 0)

        nz = _AN // _C
        nmine = (nz - s + _NS - 1) // _NS

        def _zc(k, carry):
            r0 = (s + k * _NS) * _C
            pltpu.async_copy(abuf0, accm.at[pl.ds(r0, _C)], sem1)
            pltpu.async_copy(zdeg, accd.at[pl.ds(r0, _C)], sem2)
            return carry
        lax.fori_loop(0, nmine, _zc, 0)

    with jax.named_scope("p_flags"):
        ione16 = jnp.ones((16,), jnp.int32)

        def _fz(g, carry):
            flagsv[pl.ds(g * 16, 16)] = izero16
            return carry
        lax.fori_loop(0, (_N + 16) // 16, _fz, 0)
        pltpu.sync_copy(tgth, tgtb)

        def _fb(g, carry):
            t16 = tgtb[pl.ds(g * 16, 16)]
            plsc.store_scatter(flagsv, [t16], ione16)
            return carry
        lax.fori_loop(0, _B // 16, _fb, 0)

        def _zd(k, carry):
            r0 = (s + k * _NS) * _C
            pltpu.make_async_copy(abuf0, accm.at[pl.ds(r0, _C)], sem1).wait()
            pltpu.make_async_copy(zdeg, accd.at[pl.ds(r0, _C)], sem2).wait()
            return carry
        lax.fori_loop(0, nmine, _zd, 0)

    with jax.named_scope("p_barrier1"):
        plsc.subcore_barrier()

    def _issue_g(k, ringoff):
        ringoff = pl.multiple_of(ringoff, _C)
        pltpu.async_copy(table1.at[csrc.at[pl.ds(ringoff, _C)]],
                         abufs[k], gsems[k])
        pltpu.async_copy(table2.at[crel.at[pl.ds(ringoff, _C)]],
                         bbufs[k], gsems[k])
        for g in range(_C // 16):
            dstvs[k][pl.ds(g * 16, 16)] = cdst[pl.ds(ringoff + g * 16, 16)]

    def _wait_g(k):
        pltpu.make_async_copy(table1.at[csrc.at[pl.ds(0, _C)]],
                              abufs[k], gsems[k]).wait()
        pltpu.make_async_copy(table2.at[crel.at[pl.ds(0, _C)]],
                              bbufs[k], gsems[k]).wait()

    def _compute(k):
        a_, b_ = abufs[k], bbufs[k]

        def _erow(e, carry):
            for g in range(_H // 16):
                a = a_[e, pl.ds(g * 16, 16)]
                b = b_[e, pl.ds(g * 16, 16)]
                a_[e, pl.ds(g * 16, 16)] = jnp.maximum(a + b, 0.0)
            return carry
        lax.fori_loop(0, _C, _erow, 0)

    def _scatter(k):
        pltpu.sync_copy(abufs[k], accm.at[dstvs[k]], add=True)
        pltpu.sync_copy(onesb, accd.at[dstvs[k]], add=True)

    def _chunk_sync(ringoff):
        _issue_g(0, ringoff)
        _wait_g(0)
        _compute(0)
        _scatter(0)

    base = wid * _EPW

    def _span(sp, carry):
        off = base + sp * _SP
        c1 = pltpu.async_copy(eidx.at[0, pl.ds(off, _SP)], rsrc, sem1)
        c2 = pltpu.async_copy(eidx.at[1, pl.ds(off, _SP)], rdst, sem2)
        c3 = pltpu.async_copy(relh.at[pl.ds(off, _SP)], rrel, sem3)
        c1.wait()
        c2.wait()
        c3.wait()

        def _super(su, carry2):
            cntv, po = carry2

            def _scan(g, cv):
                d16 = rdst[pl.ds(g * 16, 16)]
                fw = plsc.load_gather(flagsv, [d16])
                keep = fw > 0
                cum = plsc.cumsum(jnp.where(keep, 1, 0))
                pos = (cv + cum - 1) & _MASK
                plsc.store_scatter(csrc, [pos], rsrc[pl.ds(g * 16, 16)],
                                   mask=keep)
                plsc.store_scatter(cdst, [pos], d16, mask=keep)
                plsc.store_scatter(crel, [pos], rrel[pl.ds(g * 16, 16)],
                                   mask=keep)
                return cv + plsc.all_reduce_population_count(keep)
            cntv = lax.fori_loop(su * (_SP // _NSUP // 16),
                                 (su + 1) * (_SP // _NSUP // 16),
                                 _scan, cntv)

            cnt = jnp.max(cntv)
            nd = jnp.maximum((cnt - po - _THR + _C - 1) // _C, 0)

            def _dr(i, carry3):
                _chunk_sync((po + i * _C) & _MASK)
                return carry3
            lax.fori_loop(0, nd, _dr, 0)
            return (cntv, po + nd * _C)
        return lax.fori_loop(0, _NSUP, _super, carry)

    with jax.named_scope("p_scan"):
        cntv, po = lax.fori_loop(0, _NSPAN, _span,
                                 (jnp.zeros((16,), jnp.int32), jnp.int32(0)))
        cnt = jnp.max(cntv)

    with jax.named_scope("p_drain"):
        ntot = cnt - po
        nchunks = ((ntot + 2 * _C - 1) // (2 * _C)) * 2
        pend = po + nchunks * _C
        dumd = jnp.full((16,), _DUMMY, jnp.int32)
        for g in range(2 * _C // 16):
            p16 = cnt + g * 16 + lanes
            padm = p16 < pend
            pr = p16 & _MASK
            plsc.store_scatter(csrc, [pr], izero16, mask=padm)
            plsc.store_scatter(cdst, [pr], dumd, mask=padm)
            plsc.store_scatter(crel, [pr], izero16, mask=padm)

        npairs = nchunks // 2

        @pl.when(nchunks > 0)
        def _():
            _issue_g(0, po & _MASK)

        def _pair(j2, carry):
            p0 = po + 2 * j2 * _C
            _issue_g(1, (p0 + _C) & _MASK)
            _wait_g(0)
            _compute(0)
            _scatter(0)

            @pl.when(j2 + 1 < npairs)
            def _():
                _issue_g(0, (p0 + 2 * _C) & _MASK)
            _wait_g(1)
            _compute(1)
            _scatter(1)
            return carry
        lax.fori_loop(0, npairs, _pair, 0)

    with jax.named_scope("p_barrier2"):
        plsc.subcore_barrier()

    with jax.named_scope("p_out"):
        tb = s * _TPS
        pltpu.sync_copy(tgth.at[pl.ds(tb, _TPS)], tgtv)
        t0 = tgtv.at[pl.ds(0, 32)]
        t1 = tgtv.at[pl.ds(32, 32)]
        g0 = pltpu.async_copy(accm.at[t0], abuf0, sem1)
        g1 = pltpu.async_copy(accm.at[t1], abuf1, sem2)
        gd0 = pltpu.async_copy(accd.at[t0], zdeg, sem3)
        gd1 = pltpu.async_copy(accd.at[t1], onesb, sem4)
        g0.wait()
        w0 = pltpu.async_copy(abuf0, part.at[c, pl.ds(tb, 32)], sem1)
        g1.wait()
        w1 = pltpu.async_copy(abuf1, part.at[c, pl.ds(tb + 32, 32)], sem2)
        gd0.wait()
        wd0 = pltpu.async_copy(zdeg, degpart.at[c, pl.ds(tb, 32)], sem3)
        gd1.wait()
        wd1 = pltpu.async_copy(onesb, degpart.at[c, pl.ds(tb + 32, 32)], sem4)

        @pl.when(c == 0)
        def _():
            e0 = pltpu.async_copy(enth.at[t0], bbuf0, sem1)
            e1 = pltpu.async_copy(enth.at[t1], bbuf1, sem2)
            e0.wait()
            pltpu.async_copy(bbuf0, enttgt.at[pl.ds(tb, 32)], sem1).wait()
            e1.wait()
            pltpu.async_copy(bbuf1, enttgt.at[pl.ds(tb + 32, 32)],
                             sem2).wait()
        w0.wait()
        w1.wait()
        wd0.wait()
        wd1.wait()


def _sc_edge(table1, table2, eidx, rels, tgt, ent):
    mesh = plsc.VectorSubcoreMesh(core_axis_name="c", subcore_axis_name="s")
    fn = pl.kernel(
        _sc_body,
        out_type=(
            jax.ShapeDtypeStruct((_NC, _B, _H), jnp.float32),
            jax.ShapeDtypeStruct((_NC, _B, 16), jnp.float32),
            jax.ShapeDtypeStruct((_B, _H), jnp.float32),
        ),
        mesh=mesh,
        compiler_params=pltpu.CompilerParams(use_tc_tiling_on_sc=False,
                                             needs_layout_passes=False),
        scratch_types=[
            pltpu.VMEM_SHARED((_AN, _H), jnp.float32),
            pltpu.VMEM_SHARED((_AN, 16), jnp.float32),
            pltpu.VMEM((_SP,), jnp.int32),
            pltpu.VMEM((_SP,), jnp.int32),
            pltpu.VMEM((_SP,), jnp.int32),
            pltpu.VMEM((_CAP,), jnp.int32),
            pltpu.VMEM((_CAP,), jnp.int32),
            pltpu.VMEM((_CAP,), jnp.int32),
            pltpu.VMEM((_N + 16, ), jnp.int32),
            pltpu.VMEM((_B,), jnp.int32),
            pltpu.VMEM((_C, _H), jnp.float32),
            pltpu.VMEM((_C, _H), jnp.float32),
            pltpu.VMEM((_C, _H), jnp.float32),
            pltpu.VMEM((_C, _H), jnp.float32),
            pltpu.VMEM((_C, 16), jnp.float32),
            pltpu.VMEM((_C, 16), jnp.float32),
            pltpu.VMEM((_C,), jnp.int32),
            pltpu.VMEM((_C,), jnp.int32),
            pltpu.VMEM((_TPS,), jnp.int32),
            pltpu.SemaphoreType.DMA,
            pltpu.SemaphoreType.DMA,
            pltpu.SemaphoreType.DMA,
            pltpu.SemaphoreType.DMA,
        ],
    )
    return fn(table1, table2, eidx, rels, tgt, ent)


def _fin_body(ap_ref, dp_ref, et_ref, ge_ref, ws_ref, fc_ref, o_ref):
    agg = ap_ref[0] + ap_ref[1]
    degs = dp_ref[0] + dp_ref[1]
    deg = jnp.maximum(degs[:, 0:1], 1.0)
    selfloop = jnp.dot(et_ref[...], ws_ref[...],
                       preferred_element_type=jnp.float32)
    h = jnp.maximum(agg / deg + selfloop, 0.0)
    fc_a = fc_ref[:, 0:_H]
    fc_b = fc_ref[:, _H:2 * _H]
    o_ref[...] = (
        lax.dot_general(h, fc_a, (((1,), (1,)), ((), ())),
                        preferred_element_type=jnp.float32)
        + lax.dot_general(ge_ref[...], fc_b, (((1,), (1,)), ((), ())),
                          preferred_element_type=jnp.float32)
    )


def _finalize(ap, dp, enttgt, global_emb, w_self, fc_w):
    return pl.pallas_call(
        _fin_body,
        out_shape=jax.ShapeDtypeStruct((_B, _H), jnp.float32),
    )(ap, dp, enttgt, global_emb, w_self, fc_w)


def kernel(ent_embeds, rel_embeds, edge_index, edge_rel, target_idx,
           global_emb, W_msg, W_self, fc_W):
    table1, table2 = _make_tables(ent_embeds, rel_embeds, W_msg)
    part, degpart, enttgt = _sc_edge(table1, table2, edge_index, edge_rel,
                                     target_idx, ent_embeds)
    return _finalize(part, degpart, enttgt, global_emb, W_self, fc_W)
